# Initial kernel scaffold; baseline (speedup 1.0000x reference)
import jax
import jax.numpy as jnp
from jax.experimental import pallas as pl


def kernel(int_feature, float_feature, edge_attr, node_mask, edge_mask, x, params):
    raise NotImplementedError("write your pallas kernel here")


# speedup vs baseline: 8.4088x; 8.4088x over previous
import math

import jax
import jax.numpy as jnp
from jax.experimental import pallas as pl
from jax.experimental.pallas import tpu as pltpu

_LOG_2PI = math.log(2.0 * math.pi)


def _vdq_kernel(
    h0_ref, x_ref, easq_ref, eaf_ref, emsq_ref, emf_ref, nm_ref, int_ref,
    heps_ref, eeps_ref,
    embW_ref, embb_ref,
    W1a_ref, W1b_ref, wr_ref, we_ref, b1_ref,
    W2_ref, b2_ref, cW1_ref, cb1_ref, cW2_ref, cb2_ref,
    nW1a_ref, nW1b_ref, nb1_ref, nW2_ref, nb2_ref,
    outW_ref, outb_ref, eoutW_ref, eoutb_ref,
    vint_ref, vedge_ref, logq_ref,
):
    silu = jax.nn.silu
    n = x_ref.shape[1]
    H = embW_ref.shape[1]

    nm = nm_ref[0]
    emsq = emsq_ref[0]
    emf = emf_ref[0]
    x = x_ref[0] * nm
    h0 = h0_ref[0] * nm
    eam = easq_ref[0] * emsq

    h = jnp.dot(h0, embW_ref[...], preferred_element_type=jnp.float32) + embb_ref[...]

    n_layers = W1a_ref.shape[0]
    m = None
    for l in range(n_layers):
        sq = jnp.sum(x * x, axis=1, keepdims=True)
        G = jax.lax.dot_general(x, x, (((1,), (1,)), ((), ())),
                                preferred_element_type=jnp.float32)
        radial = sq + sq.T - 2.0 * G

        A = jnp.dot(h, W1a_ref[l], preferred_element_type=jnp.float32) + b1_ref[l]
        B = jnp.dot(h, W1b_ref[l], preferred_element_type=jnp.float32)
        pre = (A[:, None, :] + B[None, :, :]
               + radial[:, :, None] * wr_ref[l][None]
               + eam[:, :, None] * we_ref[l][None])
        t = silu(pre).reshape(n * n, H)
        m = silu(jnp.dot(t, W2_ref[l], preferred_element_type=jnp.float32)
                 + b2_ref[l]) * emf

        c1 = silu(jnp.dot(m, cW1_ref[l], preferred_element_type=jnp.float32)
                  + cb1_ref[l])
        cw = jnp.dot(c1, cW2_ref[l], preferred_element_type=jnp.float32) + cb2_ref[l]
        cwm = cw.reshape(n, n) * emsq
        srow = jnp.sum(cwm, axis=1, keepdims=True)
        x = (x + x * srow
             - jnp.dot(cwm, x, preferred_element_type=jnp.float32)) * nm

        agg = jnp.sum(m.reshape(n, n, H), axis=1)
        hu = silu(jnp.dot(h, nW1a_ref[l], preferred_element_type=jnp.float32)
                  + jnp.dot(agg, nW1b_ref[l], preferred_element_type=jnp.float32)
                  + nb1_ref[l])
        h = (h + jnp.dot(hu, nW2_ref[l], preferred_element_type=jnp.float32)
             + nb2_ref[l]) * nm

    hf = (jnp.dot(h, outW_ref[...], preferred_element_type=jnp.float32)
          + outb_ref[...]) * nm
    ef = jnp.dot(m, eoutW_ref[...], preferred_element_type=jnp.float32) + eoutb_ref[...]

    k = hf.shape[1] // 2
    h_mu, h_ls = hf[:, :k], hf[:, k:]
    e_mu, e_ls = ef[:, 0:1], ef[:, 1:2]
    heps = heps_ref[0] * nm
    eeps = eeps_ref[0] * emf
    h_u = h_mu + heps * jnp.exp(h_ls)
    e_u = e_mu + eeps * jnp.exp(e_ls)

    lsg = jax.nn.log_sigmoid
    u_int = jax.nn.sigmoid(h_u)
    u_edge = jax.nn.sigmoid(e_u)
    vint_ref[0] = (int_ref[0] + u_int) * nm
    vedge_ref[0] = (eaf_ref[0] + u_edge) * emf

    h_logq = jnp.sum(nm * (-0.5 * heps * heps - 0.5 * _LOG_2PI))
    e_logq = jnp.sum(emf * (-0.5 * eeps * eeps - 0.5 * _LOG_2PI))
    h_ldj = jnp.sum(h_ls)
    e_ldj = jnp.sum(e_ls)
    ldj_int = jnp.sum(nm * (lsg(h_u) + lsg(-h_u)))
    ldj_edge = jnp.sum(emf * (lsg(e_u) + lsg(-e_u)))
    total = h_logq - h_ldj + e_logq - e_ldj - ldj_int - ldj_edge
    logq_ref[...] = total.reshape(1, 1, 1)


def kernel(int_feature, float_feature, edge_attr, node_mask, edge_mask, x, params):
    f32 = jnp.float32
    bs, n, dx = x.shape
    nf_int = int_feature.shape[-1]
    nf = nf_int + float_feature.shape[-1]
    H = params["emb_in_W"].shape[1]
    L = params["layers"]
    nl = len(L)

    h0 = jnp.concatenate([int_feature, float_feature], axis=2)
    easq = edge_attr.reshape(bs, n, n)
    eaf = edge_attr.reshape(bs, n * n, 1)
    emsq = edge_mask.reshape(bs, n, n)
    emf = edge_mask.reshape(bs, n * n, 1)
    nm = node_mask.reshape(bs, n, 1)

    kk = jax.random.key(1234)
    k1, k2 = jax.random.split(kk)
    h_eps = jax.random.normal(k1, (bs, n, nf_int), dtype=f32)
    e_eps = jax.random.normal(k2, (bs, n * n, 1), dtype=f32)

    stk = lambda name: jnp.stack([lp[name] for lp in L])
    W1a = stk("edge_W1")[:, :H, :]
    W1b = stk("edge_W1")[:, H:2 * H, :]
    wr = stk("edge_W1")[:, 2 * H:2 * H + 1, :]
    we = stk("edge_W1")[:, 2 * H + 1:2 * H + 2, :]
    b1 = stk("edge_b1").reshape(nl, 1, H)
    W2 = stk("edge_W2")
    b2 = stk("edge_b2").reshape(nl, 1, H)
    cW1 = stk("coord_W1")
    cb1 = stk("coord_b1").reshape(nl, 1, H)
    cW2 = stk("coord_W2")
    cb2 = stk("coord_b2").reshape(nl, 1, 1)
    nW1a = stk("node_W1")[:, :H, :]
    nW1b = stk("node_W1")[:, H:, :]
    nb1 = stk("node_b1").reshape(nl, 1, H)
    nW2 = stk("node_W2")
    nb2 = stk("node_b2").reshape(nl, 1, H)
    embW = params["emb_in_W"]
    embb = params["emb_in_b"].reshape(1, H)
    outW = params["emb_out_W"]
    outb = params["emb_out_b"].reshape(1, -1)
    eoutW = params["edge_out_W"]
    eoutb = params["edge_out_b"].reshape(1, -1)

    per_graph = lambda shp: pl.BlockSpec((1,) + shp, lambda b: (b,) + (0,) * len(shp))
    whole = lambda a: pl.BlockSpec(a.shape, lambda b: (0,) * a.ndim)

    in_specs = [
        per_graph((n, nf)),
        per_graph((n, dx)),
        per_graph((n, n)),
        per_graph((n * n, 1)),
        per_graph((n, n)),
        per_graph((n * n, 1)),
        per_graph((n, 1)),
        per_graph((n, nf_int)),
        per_graph((n, nf_int)),
        per_graph((n * n, 1)),
    ] + [whole(a) for a in (embW, embb, W1a, W1b, wr, we, b1, W2, b2,
                            cW1, cb1, cW2, cb2, nW1a, nW1b, nb1, nW2, nb2,
                            outW, outb, eoutW, eoutb)]

    out_specs = [
        per_graph((n, nf_int)),
        per_graph((n * n, 1)),
        per_graph((1, 1)),
    ]
    out_shapes = [
        jax.ShapeDtypeStruct((bs, n, nf_int), f32),
        jax.ShapeDtypeStruct((bs, n * n, 1), f32),
        jax.ShapeDtypeStruct((bs, 1, 1), f32),
    ]

    v_int, v_edge, logq = pl.pallas_call(
        _vdq_kernel,
        grid=(bs,),
        in_specs=in_specs,
        out_specs=out_specs,
        out_shape=out_shapes,
        compiler_params=pltpu.CompilerParams(
            dimension_semantics=("parallel",)),
    )(h0, x, easq, eaf, emsq, emf, nm, int_feature, h_eps, e_eps,
      embW, embb, W1a, W1b, wr, we, b1, W2, b2,
      cW1, cb1, cW2, cb2, nW1a, nW1b, nb1, nW2, nb2,
      outW, outb, eoutW, eoutb)

    return (v_int, float_feature, v_edge.reshape(bs * n * n, 1),
            logq.reshape(bs))


# speedup vs baseline: 8.9798x; 1.0679x over previous
import math

import jax
import jax.numpy as jnp
from jax.experimental import pallas as pl
from jax.experimental.pallas import tpu as pltpu

_LOG_2PI = math.log(2.0 * math.pi)


def _vdq_kernel(
    h0_ref, x_ref, easq_ref, eaf_ref, emsq_ref, emf_ref, nm_ref, int_ref,
    heps_ref, eeps_ref,
    embW_ref, embb_ref,
    W1a_ref, W1b_ref, wr_ref, we_ref, b1_ref,
    W2_ref, b2_ref, cW1_ref, cb1_ref, cW2_ref, cb2_ref,
    nW1a_ref, nW1b_ref, nb1_ref, nW2_ref, nb2_ref,
    outW_ref, outb_ref, eoutW_ref, eoutb_ref,
    vint_ref, vedge_ref, logq_ref,
):
    silu = jax.nn.silu
    f32 = jnp.float32
    n = x_ref.shape[1]
    nn = n * n
    H2 = W2_ref.shape[1]
    H = H2 // 2
    ni = int_ref.shape[2]

    nm0 = nm_ref[0]
    nm1 = nm_ref[1]
    nmH = jnp.concatenate([jnp.broadcast_to(nm0, (n, H)),
                           jnp.broadcast_to(nm1, (n, H))], axis=1)
    emf0 = emf_ref[0]
    emf1 = emf_ref[1]
    emf2 = jnp.concatenate([emf0, emf1], axis=1)
    emH = jnp.concatenate([jnp.broadcast_to(emf0, (nn, H)),
                           jnp.broadcast_to(emf1, (nn, H))], axis=1)
    emsq0 = emsq_ref[0]
    emsq1 = emsq_ref[1]
    eaf2 = jnp.concatenate([eaf_ref[0], eaf_ref[1]], axis=1)
    eam0 = easq_ref[0] * emsq0
    eam1 = easq_ref[1] * emsq1

    x0 = x_ref[0] * nm0
    x1 = x_ref[1] * nm1
    h0cat = jnp.concatenate([h0_ref[0] * nm0, h0_ref[1] * nm1], axis=1)
    h = jnp.dot(h0cat, embW_ref[...], preferred_element_type=f32) + embb_ref[...]

    n_layers = W1a_ref.shape[0]
    m = None
    for l in range(n_layers):
        sq0 = jnp.sum(x0 * x0, axis=1, keepdims=True)
        G0 = jax.lax.dot_general(x0, x0, (((1,), (1,)), ((), ())),
                                 preferred_element_type=f32)
        r0 = sq0 + sq0.T - 2.0 * G0
        sq1 = jnp.sum(x1 * x1, axis=1, keepdims=True)
        G1 = jax.lax.dot_general(x1, x1, (((1,), (1,)), ((), ())),
                                 preferred_element_type=f32)
        r1 = sq1 + sq1.T - 2.0 * G1
        wrl = wr_ref[l][None]
        wel = we_ref[l][None]
        D0 = r0[:, :, None] * wrl + eam0[:, :, None] * wel
        D1 = r1[:, :, None] * wrl + eam1[:, :, None] * wel

        A = jnp.dot(h, W1a_ref[l], preferred_element_type=f32) + b1_ref[l]
        B = jnp.dot(h, W1b_ref[l], preferred_element_type=f32)
        pre = (A[:, None, :] + B[None, :, :]
               + jnp.concatenate([D0, D1], axis=2))
        t = silu(pre).reshape(nn, H2)
        m = silu(jnp.dot(t, W2_ref[l], preferred_element_type=f32)
                 + b2_ref[l]) * emH

        c1 = silu(jnp.dot(m, cW1_ref[l], preferred_element_type=f32)
                  + cb1_ref[l])
        cw = jnp.dot(c1, cW2_ref[l], preferred_element_type=f32) + cb2_ref[l]
        cwm0 = cw[:, 0:1].reshape(n, n) * emsq0
        cwm1 = cw[:, 1:2].reshape(n, n) * emsq1
        x0 = (x0 + x0 * jnp.sum(cwm0, axis=1, keepdims=True)
              - jnp.dot(cwm0, x0, preferred_element_type=f32)) * nm0
        x1 = (x1 + x1 * jnp.sum(cwm1, axis=1, keepdims=True)
              - jnp.dot(cwm1, x1, preferred_element_type=f32)) * nm1

        agg = jnp.sum(m.reshape(n, n, H2), axis=1)
        hu = silu(jnp.dot(h, nW1a_ref[l], preferred_element_type=f32)
                  + jnp.dot(agg, nW1b_ref[l], preferred_element_type=f32)
                  + nb1_ref[l])
        h = (h + jnp.dot(hu, nW2_ref[l], preferred_element_type=f32)
             + nb2_ref[l]) * nmH

    nmNi2 = jnp.concatenate([jnp.broadcast_to(nm0, (n, ni)),
                             jnp.broadcast_to(nm1, (n, ni))], axis=1)
    nm16 = jnp.concatenate([nmNi2, nmNi2], axis=1)
    hf = (jnp.dot(h, outW_ref[...], preferred_element_type=f32)
          + outb_ref[...]) * nm16
    h_mu = hf[:, :2 * ni]
    h_ls = hf[:, 2 * ni:]
    heps2 = jnp.concatenate([heps_ref[0], heps_ref[1]], axis=1) * nmNi2
    h_u = h_mu + heps2 * jnp.exp(h_ls)
    u_int = jax.nn.sigmoid(h_u)
    vint_ref[0] = (int_ref[0] + u_int[:, :ni]) * nm0
    vint_ref[1] = (int_ref[1] + u_int[:, ni:]) * nm1

    ef = jnp.dot(m, eoutW_ref[...], preferred_element_type=f32) + eoutb_ref[...]
    e_mu = ef[:, :2]
    e_ls = ef[:, 2:]
    eeps2 = jnp.concatenate([eeps_ref[0], eeps_ref[1]], axis=1) * emf2
    e_u = e_mu + eeps2 * jnp.exp(e_ls)
    u_edge = jax.nn.sigmoid(e_u)
    vedge2 = (eaf2 + u_edge) * emf2
    vedge_ref[0] = vedge2[:, 0:1]
    vedge_ref[1] = vedge2[:, 1:2]

    lsg = jax.nn.log_sigmoid
    hq = nmNi2 * (-0.5 * heps2 * heps2 - 0.5 * _LOG_2PI)
    h_logq = jnp.sum(hq.reshape(n, 2, ni), axis=(0, 2))
    h_ldj = jnp.sum(h_ls.reshape(n, 2, ni), axis=(0, 2))
    ldj_int = jnp.sum((nmNi2 * (lsg(h_u) + lsg(-h_u))).reshape(n, 2, ni),
                      axis=(0, 2))
    e_logq = jnp.sum(emf2 * (-0.5 * eeps2 * eeps2 - 0.5 * _LOG_2PI), axis=0)
    e_ldj = jnp.sum(e_ls, axis=0)
    ldj_edge = jnp.sum(emf2 * (lsg(e_u) + lsg(-e_u)), axis=0)
    total = h_logq - h_ldj + e_logq - e_ldj - ldj_int - ldj_edge
    logq_ref[...] = total.reshape(2, 1, 1)


def _bd(W):
    Z = jnp.zeros_like(W)
    top = jnp.concatenate([W, Z], axis=-1)
    bot = jnp.concatenate([Z, W], axis=-1)
    return jnp.concatenate([top, bot], axis=-2)


def kernel(int_feature, float_feature, edge_attr, node_mask, edge_mask, x, params):
    f32 = jnp.float32
    bs, n, dx = x.shape
    nn = n * n
    ni = int_feature.shape[-1]
    nf = ni + float_feature.shape[-1]
    H = params["emb_in_W"].shape[1]
    L = params["layers"]
    nl = len(L)

    h0 = jnp.concatenate([int_feature, float_feature], axis=2)
    eaf = edge_attr.reshape(bs, nn, 1)
    emsq = edge_mask.reshape(bs, n, n)
    emf = edge_mask.reshape(bs, nn, 1)
    nm = node_mask.reshape(bs, n, 1)

    kk = jax.random.key(1234)
    k1, k2 = jax.random.split(kk)
    h_eps = jax.random.normal(k1, (bs, n, ni), dtype=f32)
    e_eps = jax.random.normal(k2, (bs, nn, 1), dtype=f32)

    stk = lambda name: jnp.stack([lp[name] for lp in L])
    eW1 = stk("edge_W1")
    W1a = _bd(eW1[:, :H, :])
    W1b = _bd(eW1[:, H:2 * H, :])
    wr = eW1[:, 2 * H:2 * H + 1, :]
    we = eW1[:, 2 * H + 1:2 * H + 2, :]
    tile2 = lambda b: jnp.tile(b.reshape(nl, 1, H), (1, 1, 2))
    b1 = tile2(stk("edge_b1"))
    W2 = _bd(stk("edge_W2"))
    b2 = tile2(stk("edge_b2"))
    cW1 = _bd(stk("coord_W1"))
    cb1 = tile2(stk("coord_b1"))
    cW2 = _bd(stk("coord_W2"))
    cb2 = jnp.tile(stk("coord_b2").reshape(nl, 1, 1), (1, 1, 2))
    nW1 = stk("node_W1")
    nW1a = _bd(nW1[:, :H, :])
    nW1b = _bd(nW1[:, H:, :])
    nb1 = tile2(stk("node_b1"))
    nW2 = _bd(stk("node_W2"))
    nb2 = tile2(stk("node_b2"))
    embW = _bd(params["emb_in_W"])
    embb = jnp.tile(params["emb_in_b"].reshape(1, H), (1, 2))
    oW = params["emb_out_W"]
    zni = jnp.zeros((H, ni), f32)
    outW = jnp.concatenate([
        jnp.concatenate([oW[:, :ni], zni, oW[:, ni:], zni], axis=1),
        jnp.concatenate([zni, oW[:, :ni], zni, oW[:, ni:]], axis=1)], axis=0)
    ob = params["emb_out_b"]
    outb = jnp.concatenate([ob[:ni], ob[:ni], ob[ni:], ob[ni:]]).reshape(1, -1)
    eW = params["edge_out_W"]
    ze = jnp.zeros((H, 1), f32)
    eoutW = jnp.concatenate([
        jnp.concatenate([eW[:, :1], ze, eW[:, 1:], ze], axis=1),
        jnp.concatenate([ze, eW[:, :1], ze, eW[:, 1:]], axis=1)], axis=0)
    eb = params["edge_out_b"]
    eoutb = jnp.stack([eb[0], eb[0], eb[1], eb[1]]).reshape(1, 4)

    per_pair = lambda shp: pl.BlockSpec((2,) + shp, lambda b: (b,) + (0,) * len(shp))
    whole = lambda a: pl.BlockSpec(a.shape, lambda b: (0,) * a.ndim)

    easq = edge_attr.reshape(bs, n, n)

    in_specs = [
        per_pair((n, nf)),
        per_pair((n, dx)),
        per_pair((n, n)),
        per_pair((nn, 1)),
        per_pair((n, n)),
        per_pair((nn, 1)),
        per_pair((n, 1)),
        per_pair((n, ni)),
        per_pair((n, ni)),
        per_pair((nn, 1)),
    ] + [whole(a) for a in (embW, embb, W1a, W1b, wr, we, b1, W2, b2,
                            cW1, cb1, cW2, cb2, nW1a, nW1b, nb1, nW2, nb2,
                            outW, outb, eoutW, eoutb)]

    out_specs = [
        per_pair((n, ni)),
        per_pair((nn, 1)),
        per_pair((1, 1)),
    ]
    out_shapes = [
        jax.ShapeDtypeStruct((bs, n, ni), f32),
        jax.ShapeDtypeStruct((bs, nn, 1), f32),
        jax.ShapeDtypeStruct((bs, 1, 1), f32),
    ]

    v_int, v_edge, logq = pl.pallas_call(
        _vdq_kernel,
        grid=(bs // 2,),
        in_specs=in_specs,
        out_specs=out_specs,
        out_shape=out_shapes,
        compiler_params=pltpu.CompilerParams(
            dimension_semantics=("parallel",)),
    )(h0, x, easq, eaf, emsq, emf, nm, int_feature, h_eps, e_eps,
      embW, embb, W1a, W1b, wr, we, b1, W2, b2,
      cW1, cb1, cW2, cb2, nW1a, nW1b, nb1, nW2, nb2,
      outW, outb, eoutW, eoutb)

    return (v_int, float_feature, v_edge.reshape(bs * nn, 1),
            logq.reshape(bs))


# speedup vs baseline: 9.3393x; 1.0400x over previous
import math

import jax
import jax.numpy as jnp
from jax.experimental import pallas as pl
from jax.experimental.pallas import tpu as pltpu

_LOG_2PI = math.log(2.0 * math.pi)


def _vdq_kernel(
    h0_ref, x_ref, eaf_ref, emsq_ref, emf_ref, nm_ref, int_ref,
    heps_ref, eeps_ref,
    embW_ref, embb_ref,
    W1a_ref, W1b_ref, wr_ref, wr2_ref, we2_ref, b1_ref,
    W2_ref, b2_ref, cW1_ref, cb1_ref, cW2_ref, cb2_ref,
    nW1a_ref, nW1b_ref, nb1_ref, nW2_ref, nb2_ref,
    outW_ref, outb_ref, eoutW_ref, eoutb_ref,
    vint_ref, vedge_ref, logq_ref,
):
    silu = jax.nn.silu
    f32 = jnp.float32
    n = x_ref.shape[1]
    nn = n * n
    H2 = W2_ref.shape[1]
    H = H2 // 2
    ni = int_ref.shape[2]

    nm0 = nm_ref[0]
    nm1 = nm_ref[1]
    nmH = jnp.concatenate([jnp.broadcast_to(nm0, (n, H)),
                           jnp.broadcast_to(nm1, (n, H))], axis=1)
    emf0 = emf_ref[0]
    emf1 = emf_ref[1]
    emf2 = jnp.concatenate([emf0, emf1], axis=1)
    emH = jnp.concatenate([jnp.broadcast_to(emf0, (nn, H)),
                           jnp.broadcast_to(emf1, (nn, H))], axis=1)
    emsq0 = emsq_ref[0]
    emsq1 = emsq_ref[1]
    eaf2 = jnp.concatenate([eaf_ref[0], eaf_ref[1]], axis=1)
    Xe = eaf2 * emf2

    x0 = x_ref[0] * nm0
    x1 = x_ref[1] * nm1
    h0cat = jnp.concatenate([h0_ref[0] * nm0, h0_ref[1] * nm1], axis=1)
    h = jnp.dot(h0cat, embW_ref[...], preferred_element_type=f32) + embb_ref[...]

    n_layers = W1a_ref.shape[0]
    m = None
    for l in range(n_layers):
        sq0 = jnp.sum(x0 * x0, axis=1, keepdims=True)
        G0 = jax.lax.dot_general(x0, x0, (((1,), (1,)), ((), ())),
                                 preferred_element_type=f32)
        sq1 = jnp.sum(x1 * x1, axis=1, keepdims=True)
        G1 = jax.lax.dot_general(x1, x1, (((1,), (1,)), ((), ())),
                                 preferred_element_type=f32)
        m2wrl = -2.0 * wr_ref[l][None]
        D0 = G0[:, :, None] * m2wrl
        D1 = G1[:, :, None] * m2wrl
        sqc = jnp.concatenate([sq0, sq1], axis=1)
        S = jnp.dot(sqc, wr2_ref[l], preferred_element_type=f32)
        Deam = jnp.dot(Xe, we2_ref[l], preferred_element_type=f32)

        A = jnp.dot(h, W1a_ref[l], preferred_element_type=f32) + b1_ref[l] + S
        B = jnp.dot(h, W1b_ref[l], preferred_element_type=f32) + S
        pre = (A[:, None, :] + B[None, :, :]
               + jnp.concatenate([D0, D1], axis=2)).reshape(nn, H2) + Deam
        t = silu(pre)
        m = silu(jnp.dot(t, W2_ref[l], preferred_element_type=f32)
                 + b2_ref[l]) * emH

        c1 = silu(jnp.dot(m, cW1_ref[l], preferred_element_type=f32)
                  + cb1_ref[l])
        cw = jnp.dot(c1, cW2_ref[l], preferred_element_type=f32) + cb2_ref[l]
        cwm0 = cw[:, 0:1].reshape(n, n) * emsq0
        cwm1 = cw[:, 1:2].reshape(n, n) * emsq1
        x0 = (x0 + x0 * jnp.sum(cwm0, axis=1, keepdims=True)
              - jnp.dot(cwm0, x0, preferred_element_type=f32)) * nm0
        x1 = (x1 + x1 * jnp.sum(cwm1, axis=1, keepdims=True)
              - jnp.dot(cwm1, x1, preferred_element_type=f32)) * nm1

        agg = jnp.sum(m.reshape(n, n, H2), axis=1)
        hu = silu(jnp.dot(h, nW1a_ref[l], preferred_element_type=f32)
                  + jnp.dot(agg, nW1b_ref[l], preferred_element_type=f32)
                  + nb1_ref[l])
        h = (h + jnp.dot(hu, nW2_ref[l], preferred_element_type=f32)
             + nb2_ref[l]) * nmH

    nmNi2 = jnp.concatenate([jnp.broadcast_to(nm0, (n, ni)),
                             jnp.broadcast_to(nm1, (n, ni))], axis=1)
    nm16 = jnp.concatenate([nmNi2, nmNi2], axis=1)
    hf = (jnp.dot(h, outW_ref[...], preferred_element_type=f32)
          + outb_ref[...]) * nm16
    h_mu = hf[:, :2 * ni]
    h_ls = hf[:, 2 * ni:]
    heps2 = jnp.concatenate([heps_ref[0], heps_ref[1]], axis=1) * nmNi2
    h_u = h_mu + heps2 * jnp.exp(h_ls)
    u_int = jax.nn.sigmoid(h_u)
    vint_ref[0] = (int_ref[0] + u_int[:, :ni]) * nm0
    vint_ref[1] = (int_ref[1] + u_int[:, ni:]) * nm1

    ef = jnp.dot(m, eoutW_ref[...], preferred_element_type=f32) + eoutb_ref[...]
    e_mu = ef[:, :2]
    e_ls = ef[:, 2:]
    eeps2 = jnp.concatenate([eeps_ref[0], eeps_ref[1]], axis=1) * emf2
    e_u = e_mu + eeps2 * jnp.exp(e_ls)
    u_edge = jax.nn.sigmoid(e_u)
    vedge2 = (eaf2 + u_edge) * emf2
    vedge_ref[0] = vedge2[:, 0:1]
    vedge_ref[1] = vedge2[:, 1:2]

    lsg = jax.nn.log_sigmoid
    hq = nmNi2 * (-0.5 * heps2 * heps2 - 0.5 * _LOG_2PI)
    h_logq = jnp.sum(hq.reshape(n, 2, ni), axis=(0, 2))
    h_ldj = jnp.sum(h_ls.reshape(n, 2, ni), axis=(0, 2))
    ldj_int = jnp.sum((nmNi2 * (lsg(h_u) + lsg(-h_u))).reshape(n, 2, ni),
                      axis=(0, 2))
    e_logq = jnp.sum(emf2 * (-0.5 * eeps2 * eeps2 - 0.5 * _LOG_2PI), axis=0)
    e_ldj = jnp.sum(e_ls, axis=0)
    ldj_edge = jnp.sum(emf2 * (lsg(e_u) + lsg(-e_u)), axis=0)
    total = h_logq - h_ldj + e_logq - e_ldj - ldj_int - ldj_edge
    logq_ref[...] = total.reshape(2, 1, 1)


def _bd(W):
    Z = jnp.zeros_like(W)
    top = jnp.concatenate([W, Z], axis=-1)
    bot = jnp.concatenate([Z, W], axis=-1)
    return jnp.concatenate([top, bot], axis=-2)


def kernel(int_feature, float_feature, edge_attr, node_mask, edge_mask, x, params):
    f32 = jnp.float32
    bs, n, dx = x.shape
    nn = n * n
    ni = int_feature.shape[-1]
    nf = ni + float_feature.shape[-1]
    H = params["emb_in_W"].shape[1]
    L = params["layers"]
    nl = len(L)

    h0 = jnp.concatenate([int_feature, float_feature], axis=2)
    eaf = edge_attr.reshape(bs, nn, 1)
    emsq = edge_mask.reshape(bs, n, n)
    emf = edge_mask.reshape(bs, nn, 1)
    nm = node_mask.reshape(bs, n, 1)

    kk = jax.random.key(1234)
    k1, k2 = jax.random.split(kk)
    h_eps = jax.random.normal(k1, (bs, n, ni), dtype=f32)
    e_eps = jax.random.normal(k2, (bs, nn, 1), dtype=f32)

    stk = lambda name: jnp.stack([lp[name] for lp in L])
    eW1 = stk("edge_W1")
    W1a = _bd(eW1[:, :H, :])
    W1b = _bd(eW1[:, H:2 * H, :])
    wr = eW1[:, 2 * H:2 * H + 1, :]
    we = eW1[:, 2 * H + 1:2 * H + 2, :]
    zH = jnp.zeros((nl, 1, H), f32)
    wr2 = jnp.concatenate([jnp.concatenate([wr, zH], axis=2),
                           jnp.concatenate([zH, wr], axis=2)], axis=1)
    we2 = jnp.concatenate([jnp.concatenate([we, zH], axis=2),
                           jnp.concatenate([zH, we], axis=2)], axis=1)
    tile2 = lambda b: jnp.tile(b.reshape(nl, 1, H), (1, 1, 2))
    b1 = tile2(stk("edge_b1"))
    W2 = _bd(stk("edge_W2"))
    b2 = tile2(stk("edge_b2"))
    cW1 = _bd(stk("coord_W1"))
    cb1 = tile2(stk("coord_b1"))
    cW2 = _bd(stk("coord_W2"))
    cb2 = jnp.tile(stk("coord_b2").reshape(nl, 1, 1), (1, 1, 2))
    nW1 = stk("node_W1")
    nW1a = _bd(nW1[:, :H, :])
    nW1b = _bd(nW1[:, H:, :])
    nb1 = tile2(stk("node_b1"))
    nW2 = _bd(stk("node_W2"))
    nb2 = tile2(stk("node_b2"))
    embW = _bd(params["emb_in_W"])
    embb = jnp.tile(params["emb_in_b"].reshape(1, H), (1, 2))
    oW = params["emb_out_W"]
    zni = jnp.zeros((H, ni), f32)
    outW = jnp.concatenate([
        jnp.concatenate([oW[:, :ni], zni, oW[:, ni:], zni], axis=1),
        jnp.concatenate([zni, oW[:, :ni], zni, oW[:, ni:]], axis=1)], axis=0)
    ob = params["emb_out_b"]
    outb = jnp.concatenate([ob[:ni], ob[:ni], ob[ni:], ob[ni:]]).reshape(1, -1)
    eW = params["edge_out_W"]
    ze = jnp.zeros((H, 1), f32)
    eoutW = jnp.concatenate([
        jnp.concatenate([eW[:, :1], ze, eW[:, 1:], ze], axis=1),
        jnp.concatenate([ze, eW[:, :1], ze, eW[:, 1:]], axis=1)], axis=0)
    eb = params["edge_out_b"]
    eoutb = jnp.stack([eb[0], eb[0], eb[1], eb[1]]).reshape(1, 4)

    per_pair = lambda shp: pl.BlockSpec((2,) + shp, lambda b: (b,) + (0,) * len(shp))
    whole = lambda a: pl.BlockSpec(a.shape, lambda b: (0,) * a.ndim)

    in_specs = [
        per_pair((n, nf)),
        per_pair((n, dx)),
        per_pair((nn, 1)),
        per_pair((n, n)),
        per_pair((nn, 1)),
        per_pair((n, 1)),
        per_pair((n, ni)),
        per_pair((n, ni)),
        per_pair((nn, 1)),
    ] + [whole(a) for a in (embW, embb, W1a, W1b, wr, wr2, we2, b1, W2, b2,
                            cW1, cb1, cW2, cb2, nW1a, nW1b, nb1, nW2, nb2,
                            outW, outb, eoutW, eoutb)]

    out_specs = [
        per_pair((n, ni)),
        per_pair((nn, 1)),
        per_pair((1, 1)),
    ]
    out_shapes = [
        jax.ShapeDtypeStruct((bs, n, ni), f32),
        jax.ShapeDtypeStruct((bs, nn, 1), f32),
        jax.ShapeDtypeStruct((bs, 1, 1), f32),
    ]

    v_int, v_edge, logq = pl.pallas_call(
        _vdq_kernel,
        grid=(bs // 2,),
        in_specs=in_specs,
        out_specs=out_specs,
        out_shape=out_shapes,
        compiler_params=pltpu.CompilerParams(
            dimension_semantics=("parallel",)),
    )(h0, x, eaf, emsq, emf, nm, int_feature, h_eps, e_eps,
      embW, embb, W1a, W1b, wr, wr2, we2, b1, W2, b2,
      cW1, cb1, cW2, cb2, nW1a, nW1b, nb1, nW2, nb2,
      outW, outb, eoutW, eoutb)

    return (v_int, float_feature, v_edge.reshape(bs * nn, 1),
            logq.reshape(bs))
